# Initial kernel scaffold; baseline (speedup 1.0000x reference)
import jax
import jax.numpy as jnp
from jax.experimental import pallas as pl


def kernel(x, edge_index, W1, att_src1, att_dst1, b1, W2, att_src2, att_dst2, b2, Wc, bc):
    raise NotImplementedError("write your pallas kernel here")


# speedup vs baseline: 15.4798x; 15.4798x over previous
import functools

import jax
import jax.numpy as jnp
from jax import lax
from jax.experimental import pallas as pl
from jax.experimental.pallas import tpu as pltpu
from jax.experimental.pallas import tpu_sc as plsc

N = 10000
E = 320000
DF = 128
H1 = 8
HC = 64

NC = 2
NS = 16
L = 16

F32 = jnp.float32
I32 = jnp.int32


_RB = 1000


def _tc1_body(x_ref, w_ref, b_ref, hp_ref, at_ref):
    h = jnp.dot(x_ref[...], w_ref[0], preferred_element_type=F32)
    hp_ref[...] = h
    at_ref[0] = jnp.dot(h, b_ref[0], preferred_element_type=F32)


def _tc1(x, W1, B):
    nb = N // _RB
    return pl.pallas_call(
        _tc1_body,
        grid=(nb, H1),
        in_specs=[
            pl.BlockSpec((_RB, DF), lambda i, h: (i, 0)),
            pl.BlockSpec((1, DF, HC), lambda i, h: (h, 0, 0)),
            pl.BlockSpec((1, HC, 2), lambda i, h: (h, 0, 0)),
        ],
        out_specs=[
            pl.BlockSpec((_RB, HC), lambda i, h: (h * nb + i, 0)),
            pl.BlockSpec((1, _RB, 2), lambda i, h: (h, i, 0)),
        ],
        out_shape=[
            jax.ShapeDtypeStruct((H1 * N, HC), F32),
            jax.ShapeDtypeStruct((H1, N, 2), F32),
        ],
    )(x, W1, B)


def _tc2_body(acc1_ref, den1_ref, b1r_ref, w2_ref, t2_ref, h2_ref, at2_ref):
    total = None
    for h in range(H1):
        d = den1_ref[h, :, 0:1] + 1e-16
        h1h = jnp.maximum(acc1_ref[h] / d + b1r_ref[h], 0.0)
        t = jnp.dot(h1h, w2_ref[h], preferred_element_type=F32)
        total = t if total is None else total + t
    h2_ref[...] = total
    at2_ref[...] = jnp.dot(total, t2_ref[...], preferred_element_type=F32)


def _tc2(acc1, den1, b1r, W2r, T2):
    nb = N // _RB
    return pl.pallas_call(
        _tc2_body,
        grid=(nb,),
        in_specs=[
            pl.BlockSpec((H1, _RB, HC), lambda i: (0, i, 0)),
            pl.BlockSpec((H1, _RB, 16), lambda i: (0, i, 0)),
            pl.BlockSpec((H1, HC), lambda i: (0, 0)),
            pl.BlockSpec((H1, HC, HC), lambda i: (0, 0, 0)),
            pl.BlockSpec((HC, 2), lambda i: (0, 0)),
        ],
        out_specs=[
            pl.BlockSpec((_RB, HC), lambda i: (i, 0)),
            pl.BlockSpec((_RB, 2), lambda i: (i, 0)),
        ],
        out_shape=[
            jax.ShapeDtypeStruct((N, HC), F32),
            jax.ShapeDtypeStruct((N, 2), F32),
        ],
    )(acc1, den1, b1r, W2r, T2)


def _tc3_body(acc_ref, den_ref, b2_ref, wc_ref, bc_ref, o_ref):
    a = acc_ref[0] + acc_ref[1]
    d = den_ref[0, :, 0:1] + den_ref[1, :, 0:1] + 1e-16
    h2 = jnp.maximum(a / d + b2_ref[...], 0.0)
    o_ref[...] = jnp.dot(h2, wc_ref[...], preferred_element_type=F32) + bc_ref[...]


def _tc3(acc2, den2, b2r, Wc, bcr):
    nb = N // _RB
    return pl.pallas_call(
        _tc3_body,
        grid=(nb,),
        in_specs=[
            pl.BlockSpec((2, _RB, HC), lambda i: (0, i, 0)),
            pl.BlockSpec((2, _RB, 16), lambda i: (0, i, 0)),
            pl.BlockSpec((1, HC), lambda i: (0, 0)),
            pl.BlockSpec((HC, 1), lambda i: (0, 0)),
            pl.BlockSpec((1, 1), lambda i: (0, 0)),
        ],
        out_specs=pl.BlockSpec((_RB, 1), lambda i: (i, 0)),
        out_shape=jax.ShapeDtypeStruct((N, 1), F32),
    )(acc2, den2, b2r, Wc, bcr)


_C1 = 160
_EPT1 = E // NS
_NROW = 640


def _sc1_body(hp_ref, at1_ref, src_ref, dst_ref, acc_o, den_o,
              acc, den, table, idxs, idxd, idxg, rows, wbuf, wflat, sem):
    c = lax.axis_index("c")
    s = lax.axis_index("s")
    zf = jnp.zeros((L,), F32)
    io16 = lax.iota(I32, L)

    base_n = s * _NROW
    last = NS - 1

    for r in range(4):
        H = 4 * c + r
        pltpu.sync_copy(at1_ref.at[H], table)

        def zrow(rr, _):
            for q in range(4):
                rows[rr, pl.ds(16 * q, L)] = zf
            return 0
        lax.fori_loop(0, _C1, zrow, 0)

        def zw(rr, _):
            wbuf[rr, :] = zf
            return 0
        lax.fori_loop(0, _C1, zw, 0)

        def zcp(r0, cnt):
            pltpu.sync_copy(rows.at[pl.ds(0, cnt)], acc.at[pl.ds(r0, cnt)])
            pltpu.sync_copy(wbuf.at[pl.ds(0, cnt)], den.at[pl.ds(r0, cnt)])

        for k in range(4):
            if k < 2:
                zcp(base_n + 160 * k, 160)
            else:
                pl.when(s < last)(functools.partial(zcp, base_n + 160 * k, 160))
        pl.when(s == last)(functools.partial(zcp, N - 80, 80))
        plsc.subcore_barrier()

        offH = H * N

        def chunk(i, _):
            e0 = s * _EPT1 + i * _C1
            pltpu.sync_copy(src_ref.at[pl.ds(e0, _C1)], idxs)
            pltpu.sync_copy(dst_ref.at[pl.ds(e0, _C1)], idxd)

            def f(j, _):
                sv = idxs[pl.ds(16 * j, L)]
                idxg[pl.ds(16 * j, L)] = sv + offH
                return 0
            lax.fori_loop(0, _C1 // L, f, 0)

            cp = pltpu.async_copy(hp_ref.at[idxg], rows, sem)

            def g(j, _):
                sv = idxs[pl.ds(16 * j, L)]
                dv = idxd[pl.ds(16 * j, L)]
                a_s = plsc.load_gather(table, [sv * 2])
                a_d = plsc.load_gather(table, [dv * 2 + 1])
                e = a_s + a_d
                e = jnp.where(e >= 0.0, e, 0.2 * e)
                w = jnp.exp(e)
                wflat[pl.ds(16 * j, L)] = w
                return 0
            lax.fori_loop(0, _C1 // L, g, 0)

            cp.wait()

            def m(j, _):
                w0v = wflat[pl.ds(16 * j, L)]
                for t in range(L):
                    e = 16 * j + t
                    w0 = w0v[t]
                    for q in range(4):
                        rows[e, pl.ds(16 * q, L)] = rows[e, pl.ds(16 * q, L)] * w0
                    wbuf[e, :] = jnp.where(io16 == 0, w0, 0.0)
                return 0
            lax.fori_loop(0, _C1 // L, m, 0)

            pltpu.sync_copy(rows, acc.at[idxd], add=True)
            pltpu.sync_copy(wbuf, den.at[idxd], add=True)
            return 0
        lax.fori_loop(0, _EPT1 // _C1, chunk, 0)
        plsc.subcore_barrier()

        def wslice(r0, cnt):
            pltpu.sync_copy(acc.at[pl.ds(r0, cnt)], acc_o.at[H].at[pl.ds(r0, cnt)])
            pltpu.sync_copy(den.at[pl.ds(r0, cnt)], den_o.at[H].at[pl.ds(r0, cnt)])

        for k in range(4):
            if k < 2:
                wslice(base_n + 160 * k, 160)
            else:
                pl.when(s < last)(functools.partial(wslice, base_n + 160 * k, 160))
        pl.when(s == last)(functools.partial(wslice, N - 80, 80))


def _sc1(hp_flat, at1, src, dst):
    mesh = plsc.VectorSubcoreMesh(core_axis_name="c", subcore_axis_name="s",
                                  num_cores=NC, num_subcores=NS)
    f = pl.kernel(
        _sc1_body,
        out_type=(
            jax.ShapeDtypeStruct((H1, N, HC), F32),
            jax.ShapeDtypeStruct((H1, N, 16), F32),
        ),
        mesh=mesh,
        scratch_types=[
            pltpu.VMEM_SHARED((N, HC), F32),
            pltpu.VMEM_SHARED((N, 16), F32),
            pltpu.VMEM((N * 2,), F32),
            pltpu.VMEM((_C1,), I32),
            pltpu.VMEM((_C1,), I32),
            pltpu.VMEM((_C1,), I32),
            pltpu.VMEM((_C1, HC), F32),
            pltpu.VMEM((_C1, 16), F32),
            pltpu.VMEM((_C1,), F32),
            pltpu.SemaphoreType.DMA,
        ],
        compiler_params=pltpu.CompilerParams(needs_layout_passes=False, use_tc_tiling_on_sc=False),
    )
    return f(hp_flat, at1, src, dst)


_C2 = 400
_EPT2 = E // (NC * NS)


def _sc2_body(h2_ref, at2_ref, src_ref, dst_ref, acc_o, den_o,
              acc, den, table, idxs, idxd, rows, wbuf, wflat, sem):
    c = lax.axis_index("c")
    s = lax.axis_index("s")
    zf = jnp.zeros((L,), F32)
    io16 = lax.iota(I32, L)
    z16 = jnp.zeros((L,), I32)
    o16 = jnp.full((L,), 1, I32)

    pltpu.sync_copy(at2_ref, table)

    def zrow(r, _):
        for q in range(4):
            rows[r, pl.ds(16 * q, L)] = zf
        return 0
    lax.fori_loop(0, _C2, zrow, 0)

    def zw(r, _):
        wbuf[r, :] = zf
        return 0
    lax.fori_loop(0, _C2, zw, 0)

    base_n = s * _NROW
    last = NS - 1

    def zslice(r0, cnt):
        pltpu.sync_copy(rows.at[pl.ds(0, cnt)], acc.at[pl.ds(r0, cnt)])
        pltpu.sync_copy(wbuf.at[pl.ds(0, cnt)], den.at[pl.ds(r0, cnt)])

    zslice(base_n, 320)
    pl.when(s < last)(functools.partial(zslice, base_n + 320, 320))
    pl.when(s == last)(functools.partial(zslice, N - 80, 80))
    plsc.subcore_barrier()

    wid = c * NS + s

    def chunk(i, _):
        e0 = wid * _EPT2 + i * _C2
        pltpu.sync_copy(src_ref.at[pl.ds(e0, _C2)], idxs)
        pltpu.sync_copy(dst_ref.at[pl.ds(e0, _C2)], idxd)
        cp = pltpu.async_copy(h2_ref.at[idxs], rows, sem)

        def g(j, _):
            sv = idxs[pl.ds(16 * j, L)]
            dv = idxd[pl.ds(16 * j, L)]
            a_s = plsc.load_gather(table, [sv * 2])
            a_d = plsc.load_gather(table, [dv * 2 + 1])
            e = a_s + a_d
            e = jnp.where(e >= 0.0, e, 0.2 * e)
            w = jnp.exp(e)
            wflat[pl.ds(16 * j, L)] = w
            return 0
        lax.fori_loop(0, _C2 // L, g, 0)

        cp.wait()

        def m(j, _):
            w0v = wflat[pl.ds(16 * j, L)]
            for t in range(L):
                e = 16 * j + t
                w0 = w0v[t]
                for q in range(4):
                    rows[e, pl.ds(16 * q, L)] = rows[e, pl.ds(16 * q, L)] * w0
                wbuf[e, :] = jnp.where(io16 == 0, w0, 0.0)
            return 0
        lax.fori_loop(0, _C2 // L, m, 0)

        pltpu.sync_copy(rows, acc.at[idxd], add=True)
        pltpu.sync_copy(wbuf, den.at[idxd], add=True)
        return 0
    lax.fori_loop(0, _EPT2 // _C2, chunk, 0)
    plsc.subcore_barrier()

    def wslice(r0, cnt):
        pltpu.sync_copy(acc.at[pl.ds(r0, cnt)], acc_o.at[c].at[pl.ds(r0, cnt)])
        pltpu.sync_copy(den.at[pl.ds(r0, cnt)], den_o.at[c].at[pl.ds(r0, cnt)])

    wslice(base_n, 320)
    pl.when(s < last)(functools.partial(wslice, base_n + 320, 320))
    pl.when(s == last)(functools.partial(wslice, N - 80, 80))


def _sc2(h2pre, at2, src, dst):
    mesh = plsc.VectorSubcoreMesh(core_axis_name="c", subcore_axis_name="s",
                                  num_cores=NC, num_subcores=NS)
    f = pl.kernel(
        _sc2_body,
        out_type=(
            jax.ShapeDtypeStruct((2, N, HC), F32),
            jax.ShapeDtypeStruct((2, N, 16), F32),
        ),
        mesh=mesh,
        scratch_types=[
            pltpu.VMEM_SHARED((N, HC), F32),
            pltpu.VMEM_SHARED((N, 16), F32),
            pltpu.VMEM((N * 2,), F32),
            pltpu.VMEM((_C2,), I32),
            pltpu.VMEM((_C2,), I32),
            pltpu.VMEM((_C2, HC), F32),
            pltpu.VMEM((_C2, 16), F32),
            pltpu.VMEM((_C2,), F32),
            pltpu.SemaphoreType.DMA,
        ],
        compiler_params=pltpu.CompilerParams(needs_layout_passes=False, use_tc_tiling_on_sc=False),
    )
    return f(h2pre, at2, src, dst)


def kernel(x, edge_index, W1, att_src1, att_dst1, b1,
           W2, att_src2, att_dst2, b2, Wc, bc):
    src = edge_index[0].astype(I32)
    dst = edge_index[1].astype(I32)

    B = jnp.stack([att_src1, att_dst1], axis=-1)
    T2 = jnp.stack([att_src2[0], att_dst2[0]], axis=1)

    W1r = W1.reshape(DF, H1, HC).transpose(1, 0, 2)
    hp_flat, at1 = _tc1(x, W1r, B)
    acc1, den1 = _sc1(hp_flat, at1.reshape(H1, N * 2), src, dst)
    h2pre, at2 = _tc2(acc1, den1, b1.reshape(H1, HC), W2.reshape(H1, HC, HC), T2)
    acc2, den2 = _sc2(h2pre, at2.reshape(N * 2), src, dst)
    out = _tc3(acc2, den2, b2.reshape(1, HC), Wc, bc.reshape(1, 1))
    return jnp.squeeze(out)


# speedup vs baseline: 17.8665x; 1.1542x over previous
import functools

import jax
import jax.numpy as jnp
from jax import lax
from jax.experimental import pallas as pl
from jax.experimental.pallas import tpu as pltpu
from jax.experimental.pallas import tpu_sc as plsc

N = 10000
E = 320000
DF = 128
H1 = 8
HC = 64

NC = 2
NS = 16
L = 16

F32 = jnp.float32
I32 = jnp.int32
_PREC = None


_RB = 1000


def _tc1_body(x_ref, w_ref, b_ref, hp_ref, at_ref):
    h = jnp.dot(x_ref[...], w_ref[0], preferred_element_type=F32, precision=_PREC)
    hp_ref[...] = h
    a0 = jnp.sum(h * b_ref[0, :, 0][None, :], axis=1, keepdims=True)
    a1 = jnp.sum(h * b_ref[0, :, 1][None, :], axis=1, keepdims=True)
    at_ref[0] = jnp.concatenate([a0, a1], axis=1)


def _tc1(x, W1, B):
    nb = N // _RB
    return pl.pallas_call(
        _tc1_body,
        grid=(nb, H1),
        in_specs=[
            pl.BlockSpec((_RB, DF), lambda i, h: (i, 0)),
            pl.BlockSpec((1, DF, HC), lambda i, h: (h, 0, 0)),
            pl.BlockSpec((1, HC, 2), lambda i, h: (h, 0, 0)),
        ],
        out_specs=[
            pl.BlockSpec((_RB, HC), lambda i, h: (h * nb + i, 0)),
            pl.BlockSpec((1, _RB, 2), lambda i, h: (h, i, 0)),
        ],
        out_shape=[
            jax.ShapeDtypeStruct((H1 * N, HC), F32),
            jax.ShapeDtypeStruct((H1, N, 2), F32),
        ],
    )(x, W1, B)


def _tc2_body(acc1_ref, den1_ref, b1r_ref, w2_ref, t2_ref, h2_ref, at2_ref):
    total = None
    for h in range(H1):
        d = den1_ref[h, :, 0:1] + 1e-16
        h1h = jnp.maximum(acc1_ref[h] / d + b1r_ref[h], 0.0)
        t = jnp.dot(h1h, w2_ref[h], preferred_element_type=F32, precision=_PREC)
        total = t if total is None else total + t
    h2_ref[...] = total
    a0 = jnp.sum(total * t2_ref[:, 0][None, :], axis=1, keepdims=True)
    a1 = jnp.sum(total * t2_ref[:, 1][None, :], axis=1, keepdims=True)
    at2_ref[...] = jnp.concatenate([a0, a1], axis=1)


def _tc2(acc1, den1, b1r, W2r, T2):
    nb = N // _RB
    return pl.pallas_call(
        _tc2_body,
        grid=(nb,),
        in_specs=[
            pl.BlockSpec((H1, _RB, HC), lambda i: (0, i, 0)),
            pl.BlockSpec((H1, _RB, 16), lambda i: (0, i, 0)),
            pl.BlockSpec((H1, HC), lambda i: (0, 0)),
            pl.BlockSpec((H1, HC, HC), lambda i: (0, 0, 0)),
            pl.BlockSpec((HC, 2), lambda i: (0, 0)),
        ],
        out_specs=[
            pl.BlockSpec((_RB, HC), lambda i: (i, 0)),
            pl.BlockSpec((_RB, 2), lambda i: (i, 0)),
        ],
        out_shape=[
            jax.ShapeDtypeStruct((N, HC), F32),
            jax.ShapeDtypeStruct((N, 2), F32),
        ],
    )(acc1, den1, b1r, W2r, T2)


def _tc3_body(acc_ref, den_ref, b2_ref, wc_ref, bc_ref, o_ref):
    a = acc_ref[0] + acc_ref[1]
    d = den_ref[0, :, 0:1] + den_ref[1, :, 0:1] + 1e-16
    h2 = jnp.maximum(a / d + b2_ref[...], 0.0)
    o_ref[...] = jnp.dot(h2, wc_ref[...], preferred_element_type=F32, precision=_PREC) + bc_ref[...]


def _tc3(acc2, den2, b2r, Wc, bcr):
    nb = N // _RB
    return pl.pallas_call(
        _tc3_body,
        grid=(nb,),
        in_specs=[
            pl.BlockSpec((2, _RB, HC), lambda i: (0, i, 0)),
            pl.BlockSpec((2, _RB, 16), lambda i: (0, i, 0)),
            pl.BlockSpec((1, HC), lambda i: (0, 0)),
            pl.BlockSpec((HC, 1), lambda i: (0, 0)),
            pl.BlockSpec((1, 1), lambda i: (0, 0)),
        ],
        out_specs=pl.BlockSpec((_RB, 1), lambda i: (i, 0)),
        out_shape=jax.ShapeDtypeStruct((N, 1), F32),
    )(acc2, den2, b2r, Wc, bcr)


_C1 = 160
_EPT1 = E // NS
_NROW = 640


def _sc1_body(hp_ref, at1_ref, src_ref, dst_ref, acc_o, den_o,
              acc, den, table, idxs0, idxs1, idxd0, idxd1, idxg0, idxg1,
              rows0, rows1, wbuf, wflat, sem0, sem1):
    idxs = (idxs0, idxs1)
    idxd = (idxd0, idxd1)
    idxg = (idxg0, idxg1)
    rows = (rows0, rows1)
    sems = (sem0, sem1)
    c = lax.axis_index("c")
    s = lax.axis_index("s")
    zf = jnp.zeros((L,), F32)
    io16 = lax.iota(I32, L)

    base_n = s * _NROW
    last = NS - 1

    for r in range(4):
        H = 4 * c + r
        pltpu.sync_copy(at1_ref.at[H], table)

        def zrow(rr, _):
            for q in range(4):
                rows0[rr, pl.ds(16 * q, L)] = zf
            return 0
        lax.fori_loop(0, _C1, zrow, 0)

        def zw(rr, _):
            wbuf[rr, :] = zf
            return 0
        lax.fori_loop(0, _C1, zw, 0)

        def zcp(r0, cnt):
            pltpu.sync_copy(rows0.at[pl.ds(0, cnt)], acc.at[pl.ds(r0, cnt)])
            pltpu.sync_copy(wbuf.at[pl.ds(0, cnt)], den.at[pl.ds(r0, cnt)])

        for k in range(4):
            if k < 2:
                zcp(base_n + 160 * k, 160)
            else:
                pl.when(s < last)(functools.partial(zcp, base_n + 160 * k, 160))
        pl.when(s == last)(functools.partial(zcp, N - 80, 80))
        plsc.subcore_barrier()

        offH = H * N
        nch = _EPT1 // _C1

        def load_chunk(i, o):
            e0 = s * _EPT1 + i * _C1
            pltpu.sync_copy(src_ref.at[pl.ds(e0, _C1)], idxs[o])
            pltpu.sync_copy(dst_ref.at[pl.ds(e0, _C1)], idxd[o])

            def f(j, _):
                sv = idxs[o][pl.ds(16 * j, L)]
                idxg[o][pl.ds(16 * j, L)] = sv + offH
                return 0
            lax.fori_loop(0, _C1 // L, f, 0)
            pltpu.async_copy(hp_ref.at[idxg[o]], rows[o], sems[o])

        def process(b):
            def g(j, _):
                sv = idxs[b][pl.ds(16 * j, L)]
                dv = idxd[b][pl.ds(16 * j, L)]
                a_s = plsc.load_gather(table, [sv * 2])
                a_d = plsc.load_gather(table, [dv * 2 + 1])
                e = a_s + a_d
                e = jnp.where(e >= 0.0, e, 0.2 * e)
                wflat[pl.ds(16 * j, L)] = jnp.exp(e)
                return 0
            lax.fori_loop(0, _C1 // L, g, 0)

            pltpu.make_async_copy(hp_ref.at[pl.ds(0, _C1)], rows[b], sems[b]).wait()

            def m(j, _):
                w0v = wflat[pl.ds(16 * j, L)]
                for t in range(L):
                    e = 16 * j + t
                    w0 = w0v[t]
                    for q in range(4):
                        rows[b][e, pl.ds(16 * q, L)] = (
                            rows[b][e, pl.ds(16 * q, L)] * w0)
                    wbuf[e, :] = jnp.where(io16 == 0, w0, 0.0)
                return 0
            lax.fori_loop(0, _C1 // L, m, 0)

            pltpu.sync_copy(rows[b], acc.at[idxd[b]], add=True)
            pltpu.sync_copy(wbuf, den.at[idxd[b]], add=True)

        load_chunk(0, 0)

        def pipe(gi, _):
            load_chunk(2 * gi + 1, 1)
            process(0)
            load_chunk(2 * gi + 2, 0)
            process(1)
            return 0
        lax.fori_loop(0, (nch - 1) // 2, pipe, 0)
        process(0)
        plsc.subcore_barrier()

        def wslice(r0, cnt):
            pltpu.sync_copy(acc.at[pl.ds(r0, cnt)], acc_o.at[H].at[pl.ds(r0, cnt)])
            pltpu.sync_copy(den.at[pl.ds(r0, cnt)], den_o.at[H].at[pl.ds(r0, cnt)])

        for k in range(4):
            if k < 2:
                wslice(base_n + 160 * k, 160)
            else:
                pl.when(s < last)(functools.partial(wslice, base_n + 160 * k, 160))
        pl.when(s == last)(functools.partial(wslice, N - 80, 80))


def _sc1(hp_flat, at1, src, dst):
    mesh = plsc.VectorSubcoreMesh(core_axis_name="c", subcore_axis_name="s",
                                  num_cores=NC, num_subcores=NS)
    f = pl.kernel(
        _sc1_body,
        out_type=(
            jax.ShapeDtypeStruct((H1, N, HC), F32),
            jax.ShapeDtypeStruct((H1, N, 16), F32),
        ),
        mesh=mesh,
        scratch_types=[
            pltpu.VMEM_SHARED((N, HC), F32),
            pltpu.VMEM_SHARED((N, 16), F32),
            pltpu.VMEM((N * 2,), F32),
            pltpu.VMEM((_C1,), I32),
            pltpu.VMEM((_C1,), I32),
            pltpu.VMEM((_C1,), I32),
            pltpu.VMEM((_C1,), I32),
            pltpu.VMEM((_C1,), I32),
            pltpu.VMEM((_C1,), I32),
            pltpu.VMEM((_C1, HC), F32),
            pltpu.VMEM((_C1, HC), F32),
            pltpu.VMEM((_C1, 16), F32),
            pltpu.VMEM((_C1,), F32),
            pltpu.SemaphoreType.DMA,
            pltpu.SemaphoreType.DMA,
        ],
        compiler_params=pltpu.CompilerParams(needs_layout_passes=False, use_tc_tiling_on_sc=False),
    )
    return f(hp_flat, at1, src, dst)


_C2 = 400
_EPT2 = E // (NC * NS)


def _sc2_body(h2_ref, at2_ref, src_ref, dst_ref, acc_o, den_o,
              acc, den, table, idxs0, idxs1, idxd0, idxd1,
              rows0, rows1, wbuf, wflat, sem0, sem1):
    idxs = (idxs0, idxs1)
    idxd = (idxd0, idxd1)
    rows = (rows0, rows1)
    sems = (sem0, sem1)
    c = lax.axis_index("c")
    s = lax.axis_index("s")
    zf = jnp.zeros((L,), F32)
    io16 = lax.iota(I32, L)

    pltpu.sync_copy(at2_ref, table)

    def zrow(r, _):
        for q in range(4):
            rows0[r, pl.ds(16 * q, L)] = zf
        return 0
    lax.fori_loop(0, _C2, zrow, 0)

    def zw(r, _):
        wbuf[r, :] = zf
        return 0
    lax.fori_loop(0, _C2, zw, 0)

    base_n = s * _NROW
    last = NS - 1

    def zslice(r0, cnt):
        pltpu.sync_copy(rows0.at[pl.ds(0, cnt)], acc.at[pl.ds(r0, cnt)])
        pltpu.sync_copy(wbuf.at[pl.ds(0, cnt)], den.at[pl.ds(r0, cnt)])

    zslice(base_n, 320)
    pl.when(s < last)(functools.partial(zslice, base_n + 320, 320))
    pl.when(s == last)(functools.partial(zslice, N - 80, 80))
    plsc.subcore_barrier()

    wid = c * NS + s
    nch = _EPT2 // _C2

    def load_chunk(i, o):
        e0 = wid * _EPT2 + i * _C2
        pltpu.sync_copy(src_ref.at[pl.ds(e0, _C2)], idxs[o])
        pltpu.sync_copy(dst_ref.at[pl.ds(e0, _C2)], idxd[o])
        pltpu.async_copy(h2_ref.at[idxs[o]], rows[o], sems[o])

    def process(b):
        def g(j, _):
            sv = idxs[b][pl.ds(16 * j, L)]
            dv = idxd[b][pl.ds(16 * j, L)]
            a_s = plsc.load_gather(table, [sv * 2])
            a_d = plsc.load_gather(table, [dv * 2 + 1])
            e = a_s + a_d
            e = jnp.where(e >= 0.0, e, 0.2 * e)
            wflat[pl.ds(16 * j, L)] = jnp.exp(e)
            return 0
        lax.fori_loop(0, _C2 // L, g, 0)

        pltpu.make_async_copy(h2_ref.at[pl.ds(0, _C2)], rows[b], sems[b]).wait()

        def m(j, _):
            w0v = wflat[pl.ds(16 * j, L)]
            for t in range(L):
                e = 16 * j + t
                w0 = w0v[t]
                for q in range(4):
                    rows[b][e, pl.ds(16 * q, L)] = rows[b][e, pl.ds(16 * q, L)] * w0
                wbuf[e, :] = jnp.where(io16 == 0, w0, 0.0)
            return 0
        lax.fori_loop(0, _C2 // L, m, 0)

        pltpu.sync_copy(rows[b], acc.at[idxd[b]], add=True)
        pltpu.sync_copy(wbuf, den.at[idxd[b]], add=True)

    load_chunk(0, 0)

    def pipe(gi, _):
        load_chunk(2 * gi + 1, 1)
        process(0)
        load_chunk(2 * gi + 2, 0)
        process(1)
        return 0
    lax.fori_loop(0, (nch - 1) // 2, pipe, 0)
    process(0)
    plsc.subcore_barrier()

    def wslice(r0, cnt):
        pltpu.sync_copy(acc.at[pl.ds(r0, cnt)], acc_o.at[c].at[pl.ds(r0, cnt)])
        pltpu.sync_copy(den.at[pl.ds(r0, cnt)], den_o.at[c].at[pl.ds(r0, cnt)])

    wslice(base_n, 320)
    pl.when(s < last)(functools.partial(wslice, base_n + 320, 320))
    pl.when(s == last)(functools.partial(wslice, N - 80, 80))


def _sc2(h2pre, at2, src, dst):
    mesh = plsc.VectorSubcoreMesh(core_axis_name="c", subcore_axis_name="s",
                                  num_cores=NC, num_subcores=NS)
    f = pl.kernel(
        _sc2_body,
        out_type=(
            jax.ShapeDtypeStruct((2, N, HC), F32),
            jax.ShapeDtypeStruct((2, N, 16), F32),
        ),
        mesh=mesh,
        scratch_types=[
            pltpu.VMEM_SHARED((N, HC), F32),
            pltpu.VMEM_SHARED((N, 16), F32),
            pltpu.VMEM((N * 2,), F32),
            pltpu.VMEM((_C2,), I32),
            pltpu.VMEM((_C2,), I32),
            pltpu.VMEM((_C2,), I32),
            pltpu.VMEM((_C2,), I32),
            pltpu.VMEM((_C2, HC), F32),
            pltpu.VMEM((_C2, HC), F32),
            pltpu.VMEM((_C2, 16), F32),
            pltpu.VMEM((_C2,), F32),
            pltpu.SemaphoreType.DMA,
            pltpu.SemaphoreType.DMA,
        ],
        compiler_params=pltpu.CompilerParams(needs_layout_passes=False, use_tc_tiling_on_sc=False),
    )
    return f(h2pre, at2, src, dst)


def kernel(x, edge_index, W1, att_src1, att_dst1, b1,
           W2, att_src2, att_dst2, b2, Wc, bc):
    src = edge_index[0].astype(I32)
    dst = edge_index[1].astype(I32)

    B = jnp.stack([att_src1, att_dst1], axis=-1)
    T2 = jnp.stack([att_src2[0], att_dst2[0]], axis=1)

    W1r = W1.reshape(DF, H1, HC).transpose(1, 0, 2)
    hp_flat, at1 = _tc1(x, W1r, B)
    acc1, den1 = _sc1(hp_flat, at1.reshape(H1, N * 2), src, dst)
    h2pre, at2 = _tc2(acc1, den1, b1.reshape(H1, HC), W2.reshape(H1, HC, HC), T2)
    acc2, den2 = _sc2(h2pre, at2.reshape(N * 2), src, dst)
    out = _tc3(acc2, den2, b2.reshape(1, HC), Wc, bc.reshape(1, 1))
    return jnp.squeeze(out)


# speedup vs baseline: 19.3692x; 1.0841x over previous
import functools

import jax
import jax.numpy as jnp
from jax import lax
from jax.experimental import pallas as pl
from jax.experimental.pallas import tpu as pltpu
from jax.experimental.pallas import tpu_sc as plsc

N = 10000
E = 320000
DF = 128
H1 = 8
HC = 64

NC = 2
NS = 16
L = 16

F32 = jnp.float32
I32 = jnp.int32
_PREC = None


_RB = 1000


def _tc1_body(x_ref, w_ref, b_ref, hp_ref, at_ref):
    h = jnp.dot(x_ref[...], w_ref[0], preferred_element_type=F32, precision=_PREC)
    hp_ref[...] = h
    a0 = jnp.sum(h * b_ref[0, :, 0][None, :], axis=1, keepdims=True)
    a1 = jnp.sum(h * b_ref[0, :, 1][None, :], axis=1, keepdims=True)
    at_ref[0] = jnp.concatenate([a0, a1], axis=1)


def _tc1(x, W1, B):
    nb = N // _RB
    return pl.pallas_call(
        _tc1_body,
        grid=(nb, H1),
        in_specs=[
            pl.BlockSpec((_RB, DF), lambda i, h: (i, 0)),
            pl.BlockSpec((1, DF, HC), lambda i, h: (h, 0, 0)),
            pl.BlockSpec((1, HC, 2), lambda i, h: (h, 0, 0)),
        ],
        out_specs=[
            pl.BlockSpec((_RB, HC), lambda i, h: (h * nb + i, 0)),
            pl.BlockSpec((1, _RB, 2), lambda i, h: (h, i, 0)),
        ],
        out_shape=[
            jax.ShapeDtypeStruct((H1 * N, HC), F32),
            jax.ShapeDtypeStruct((H1, N, 2), F32),
        ],
    )(x, W1, B)


def _tc2_body(acc1_ref, den1_ref, b1r_ref, w2_ref, t2_ref, h2_ref, at2_ref):
    total = None
    for h in range(H1):
        d = den1_ref[h, :, 0:1] + 1e-16
        h1h = jnp.maximum(acc1_ref[h] / d + b1r_ref[h], 0.0)
        t = jnp.dot(h1h, w2_ref[h], preferred_element_type=F32, precision=_PREC)
        total = t if total is None else total + t
    h2_ref[...] = total
    a0 = jnp.sum(total * t2_ref[:, 0][None, :], axis=1, keepdims=True)
    a1 = jnp.sum(total * t2_ref[:, 1][None, :], axis=1, keepdims=True)
    at2_ref[...] = jnp.concatenate([a0, a1], axis=1)


def _tc2(acc1, den1, b1r, W2r, T2):
    nb = N // _RB
    return pl.pallas_call(
        _tc2_body,
        grid=(nb,),
        in_specs=[
            pl.BlockSpec((H1, _RB, HC), lambda i: (0, i, 0)),
            pl.BlockSpec((H1, _RB, 16), lambda i: (0, i, 0)),
            pl.BlockSpec((H1, HC), lambda i: (0, 0)),
            pl.BlockSpec((H1, HC, HC), lambda i: (0, 0, 0)),
            pl.BlockSpec((HC, 2), lambda i: (0, 0)),
        ],
        out_specs=[
            pl.BlockSpec((_RB, HC), lambda i: (i, 0)),
            pl.BlockSpec((_RB, 2), lambda i: (i, 0)),
        ],
        out_shape=[
            jax.ShapeDtypeStruct((N, HC), F32),
            jax.ShapeDtypeStruct((N, 2), F32),
        ],
    )(acc1, den1, b1r, W2r, T2)


def _tc3_body(acc_ref, den_ref, b2_ref, wc_ref, bc_ref, o_ref):
    a = acc_ref[0] + acc_ref[1]
    d = den_ref[0, :, 0:1] + den_ref[1, :, 0:1] + 1e-16
    h2 = jnp.maximum(a / d + b2_ref[...], 0.0)
    o_ref[...] = jnp.dot(h2, wc_ref[...], preferred_element_type=F32, precision=_PREC) + bc_ref[...]


def _tc3(acc2, den2, b2r, Wc, bcr):
    nb = N // _RB
    return pl.pallas_call(
        _tc3_body,
        grid=(nb,),
        in_specs=[
            pl.BlockSpec((2, _RB, HC), lambda i: (0, i, 0)),
            pl.BlockSpec((2, _RB, 16), lambda i: (0, i, 0)),
            pl.BlockSpec((1, HC), lambda i: (0, 0)),
            pl.BlockSpec((HC, 1), lambda i: (0, 0)),
            pl.BlockSpec((1, 1), lambda i: (0, 0)),
        ],
        out_specs=pl.BlockSpec((_RB, 1), lambda i: (i, 0)),
        out_shape=jax.ShapeDtypeStruct((N, 1), F32),
    )(acc2, den2, b2r, Wc, bcr)


_C1 = 160
_EPT1 = E // NS
_NROW = 640


def _sc1_body(hp_ref, at1_ref, src_ref, dst_ref, acc_o, den_o,
              acc, den, table, idxs0, idxs1, idxd0, idxd1, idxg0, idxg1,
              rows0, rows1, wbuf0, wbuf1, wflat, sem0, sem1, tsem0, tsem1):
    idxs = (idxs0, idxs1)
    idxd = (idxd0, idxd1)
    idxg = (idxg0, idxg1)
    rows = (rows0, rows1)
    wbuf2 = (wbuf0, wbuf1)
    wbuf = wbuf0
    sems = (sem0, sem1)
    tsems = (tsem0, tsem1)
    c = lax.axis_index("c")
    s = lax.axis_index("s")
    zf = jnp.zeros((L,), F32)
    io16 = lax.iota(I32, L)

    base_n = s * _NROW
    last = NS - 1

    for r in range(4):
        H = 4 * c + r
        pltpu.sync_copy(at1_ref.at[H], table)

        def zrow(rr, _):
            for q in range(4):
                rows0[rr, pl.ds(16 * q, L)] = zf
            return 0
        lax.fori_loop(0, _C1, zrow, 0)

        def zw(rr, _):
            wbuf[rr, :] = zf
            return 0
        lax.fori_loop(0, _C1, zw, 0)

        def zcp(r0, cnt):
            pltpu.sync_copy(rows0.at[pl.ds(0, cnt)], acc.at[pl.ds(r0, cnt)])
            pltpu.sync_copy(wbuf.at[pl.ds(0, cnt)], den.at[pl.ds(r0, cnt)])

        for k in range(4):
            if k < 2:
                zcp(base_n + 160 * k, 160)
            else:
                pl.when(s < last)(functools.partial(zcp, base_n + 160 * k, 160))
        pl.when(s == last)(functools.partial(zcp, N - 80, 80))
        plsc.subcore_barrier()

        offH = H * N
        nch = _EPT1 // _C1

        def drain_scatter(b):
            pltpu.make_async_copy(hp_ref.at[pl.ds(0, _C1)], rows[b], tsems[b]).wait()
            pltpu.make_async_copy(den_o.at[0].at[pl.ds(0, _C1)], wbuf2[b], tsems[b]).wait()

        def load_chunk(i, o, drain):
            if drain:
                drain_scatter(o)
            e0 = s * _EPT1 + i * _C1
            pltpu.sync_copy(src_ref.at[pl.ds(e0, _C1)], idxs[o])
            pltpu.sync_copy(dst_ref.at[pl.ds(e0, _C1)], idxd[o])

            def f(j, _):
                sv = idxs[o][pl.ds(16 * j, L)]
                idxg[o][pl.ds(16 * j, L)] = sv + offH
                return 0
            lax.fori_loop(0, _C1 // L, f, 0)
            pltpu.async_copy(hp_ref.at[idxg[o]], rows[o], sems[o])

        def process(b):
            def g(j, _):
                sv = idxs[b][pl.ds(16 * j, L)]
                dv = idxd[b][pl.ds(16 * j, L)]
                a_s = plsc.load_gather(table, [sv * 2])
                a_d = plsc.load_gather(table, [dv * 2 + 1])
                e = a_s + a_d
                e = jnp.where(e >= 0.0, e, 0.2 * e)
                wflat[pl.ds(16 * j, L)] = jnp.exp(e)
                return 0
            lax.fori_loop(0, _C1 // L, g, 0)

            pltpu.make_async_copy(hp_ref.at[pl.ds(0, _C1)], rows[b], sems[b]).wait()

            def m(j, _):
                w0v = wflat[pl.ds(16 * j, L)]
                for t in range(L):
                    e = 16 * j + t
                    w0 = w0v[t]
                    for q in range(4):
                        rows[b][e, pl.ds(16 * q, L)] = (
                            rows[b][e, pl.ds(16 * q, L)] * w0)
                    wbuf2[b][e, :] = jnp.where(io16 == 0, w0, 0.0)
                return 0
            lax.fori_loop(0, _C1 // L, m, 0)

            pltpu.async_copy(rows[b], acc.at[idxd[b]], tsems[b], add=True)
            pltpu.async_copy(wbuf2[b], den.at[idxd[b]], tsems[b], add=True)

        load_chunk(0, 0, False)
        load_chunk(1, 1, False)
        process(0)

        def pipe(gi, _):
            load_chunk(2 * gi + 2, 0, True)
            process(1)
            process(0)
            load_chunk(2 * gi + 3, 1, True)
            return 0
        lax.fori_loop(0, (nch - 3) // 2, pipe, 0)
        load_chunk(nch - 1, 0, True)
        process(1)
        process(0)
        drain_scatter(0)
        drain_scatter(1)
        plsc.subcore_barrier()

        def wslice(r0, cnt):
            pltpu.sync_copy(acc.at[pl.ds(r0, cnt)], acc_o.at[H].at[pl.ds(r0, cnt)])
            pltpu.sync_copy(den.at[pl.ds(r0, cnt)], den_o.at[H].at[pl.ds(r0, cnt)])

        for k in range(4):
            if k < 2:
                wslice(base_n + 160 * k, 160)
            else:
                pl.when(s < last)(functools.partial(wslice, base_n + 160 * k, 160))
        pl.when(s == last)(functools.partial(wslice, N - 80, 80))


def _sc1(hp_flat, at1, src, dst):
    mesh = plsc.VectorSubcoreMesh(core_axis_name="c", subcore_axis_name="s",
                                  num_cores=NC, num_subcores=NS)
    f = pl.kernel(
        _sc1_body,
        out_type=(
            jax.ShapeDtypeStruct((H1, N, HC), F32),
            jax.ShapeDtypeStruct((H1, N, 16), F32),
        ),
        mesh=mesh,
        scratch_types=[
            pltpu.VMEM_SHARED((N, HC), F32),
            pltpu.VMEM_SHARED((N, 16), F32),
            pltpu.VMEM((N * 2,), F32),
            pltpu.VMEM((_C1,), I32),
            pltpu.VMEM((_C1,), I32),
            pltpu.VMEM((_C1,), I32),
            pltpu.VMEM((_C1,), I32),
            pltpu.VMEM((_C1,), I32),
            pltpu.VMEM((_C1,), I32),
            pltpu.VMEM((_C1, HC), F32),
            pltpu.VMEM((_C1, HC), F32),
            pltpu.VMEM((_C1, 16), F32),
            pltpu.VMEM((_C1, 16), F32),
            pltpu.VMEM((_C1,), F32),
            pltpu.SemaphoreType.DMA,
            pltpu.SemaphoreType.DMA,
            pltpu.SemaphoreType.DMA,
            pltpu.SemaphoreType.DMA,
        ],
        compiler_params=pltpu.CompilerParams(needs_layout_passes=False, use_tc_tiling_on_sc=False),
    )
    return f(hp_flat, at1, src, dst)


_C2 = 80
_EPT2 = E // (NC * NS)


def _sc2_body(h2_ref, at2_ref, src_ref, dst_ref, acc_o, den_o,
              acc, den, table, idxs0, idxs1, idxd0, idxd1,
              rows0, rows1, wbuf0, wbuf1, wflat, sem0, sem1, tsem0, tsem1):
    idxs = (idxs0, idxs1)
    idxd = (idxd0, idxd1)
    rows = (rows0, rows1)
    wbuf2 = (wbuf0, wbuf1)
    wbuf = wbuf0
    sems = (sem0, sem1)
    tsems = (tsem0, tsem1)
    c = lax.axis_index("c")
    s = lax.axis_index("s")
    zf = jnp.zeros((L,), F32)
    io16 = lax.iota(I32, L)

    pltpu.sync_copy(at2_ref, table)

    def zrow(r, _):
        for q in range(4):
            rows0[r, pl.ds(16 * q, L)] = zf
        return 0
    lax.fori_loop(0, _C2, zrow, 0)

    def zw(r, _):
        wbuf[r, :] = zf
        return 0
    lax.fori_loop(0, _C2, zw, 0)

    base_n = s * _NROW
    last = NS - 1

    def zslice(r0, cnt):
        for k in range(cnt // _C2):
            pltpu.sync_copy(rows0, acc.at[pl.ds(r0 + k * _C2, _C2)])
            pltpu.sync_copy(wbuf, den.at[pl.ds(r0 + k * _C2, _C2)])

    zslice(base_n, 320)
    pl.when(s < last)(functools.partial(zslice, base_n + 320, 320))
    pl.when(s == last)(functools.partial(zslice, N - 80, 80))
    plsc.subcore_barrier()

    wid = c * NS + s
    nch = _EPT2 // _C2

    def drain_scatter(b):
        pltpu.make_async_copy(h2_ref.at[pl.ds(0, _C2)], rows[b], tsems[b]).wait()
        pltpu.make_async_copy(den_o.at[0].at[pl.ds(0, _C2)], wbuf2[b], tsems[b]).wait()

    def load_chunk(i, o, drain):
        if drain:
            drain_scatter(o)
        e0 = wid * _EPT2 + i * _C2
        pltpu.sync_copy(src_ref.at[pl.ds(e0, _C2)], idxs[o])
        pltpu.sync_copy(dst_ref.at[pl.ds(e0, _C2)], idxd[o])
        pltpu.async_copy(h2_ref.at[idxs[o]], rows[o], sems[o])

    def process(b):
        def g(j, _):
            sv = idxs[b][pl.ds(16 * j, L)]
            dv = idxd[b][pl.ds(16 * j, L)]
            a_s = plsc.load_gather(table, [sv * 2])
            a_d = plsc.load_gather(table, [dv * 2 + 1])
            e = a_s + a_d
            e = jnp.where(e >= 0.0, e, 0.2 * e)
            wflat[pl.ds(16 * j, L)] = jnp.exp(e)
            return 0
        lax.fori_loop(0, _C2 // L, g, 0)

        pltpu.make_async_copy(h2_ref.at[pl.ds(0, _C2)], rows[b], sems[b]).wait()

        def m(j, _):
            w0v = wflat[pl.ds(16 * j, L)]
            for t in range(L):
                e = 16 * j + t
                w0 = w0v[t]
                for q in range(4):
                    rows[b][e, pl.ds(16 * q, L)] = rows[b][e, pl.ds(16 * q, L)] * w0
                wbuf2[b][e, :] = jnp.where(io16 == 0, w0, 0.0)
            return 0
        lax.fori_loop(0, _C2 // L, m, 0)

        pltpu.async_copy(rows[b], acc.at[idxd[b]], tsems[b], add=True)
        pltpu.async_copy(wbuf2[b], den.at[idxd[b]], tsems[b], add=True)

    load_chunk(0, 0, False)
    load_chunk(1, 1, False)
    process(0)

    def pipe(gi, _):
        load_chunk(2 * gi + 2, 0, True)
        process(1)
        process(0)
        load_chunk(2 * gi + 3, 1, True)
        return 0
    lax.fori_loop(0, (nch - 3) // 2, pipe, 0)
    load_chunk(nch - 1, 0, True)
    process(1)
    process(0)
    drain_scatter(0)
    drain_scatter(1)
    plsc.subcore_barrier()

    def wslice(r0, cnt):
        pltpu.sync_copy(acc.at[pl.ds(r0, cnt)], acc_o.at[c].at[pl.ds(r0, cnt)])
        pltpu.sync_copy(den.at[pl.ds(r0, cnt)], den_o.at[c].at[pl.ds(r0, cnt)])

    wslice(base_n, 320)
    pl.when(s < last)(functools.partial(wslice, base_n + 320, 320))
    pl.when(s == last)(functools.partial(wslice, N - 80, 80))


def _sc2(h2pre, at2, src, dst):
    mesh = plsc.VectorSubcoreMesh(core_axis_name="c", subcore_axis_name="s",
                                  num_cores=NC, num_subcores=NS)
    f = pl.kernel(
        _sc2_body,
        out_type=(
            jax.ShapeDtypeStruct((2, N, HC), F32),
            jax.ShapeDtypeStruct((2, N, 16), F32),
        ),
        mesh=mesh,
        scratch_types=[
            pltpu.VMEM_SHARED((N, HC), F32),
            pltpu.VMEM_SHARED((N, 16), F32),
            pltpu.VMEM((N * 2,), F32),
            pltpu.VMEM((_C2,), I32),
            pltpu.VMEM((_C2,), I32),
            pltpu.VMEM((_C2,), I32),
            pltpu.VMEM((_C2,), I32),
            pltpu.VMEM((_C2, HC), F32),
            pltpu.VMEM((_C2, HC), F32),
            pltpu.VMEM((_C2, 16), F32),
            pltpu.VMEM((_C2, 16), F32),
            pltpu.VMEM((_C2,), F32),
            pltpu.SemaphoreType.DMA,
            pltpu.SemaphoreType.DMA,
            pltpu.SemaphoreType.DMA,
            pltpu.SemaphoreType.DMA,
        ],
        compiler_params=pltpu.CompilerParams(needs_layout_passes=False, use_tc_tiling_on_sc=False),
    )
    return f(h2pre, at2, src, dst)


def kernel(x, edge_index, W1, att_src1, att_dst1, b1,
           W2, att_src2, att_dst2, b2, Wc, bc):
    src = edge_index[0].astype(I32)
    dst = edge_index[1].astype(I32)

    B = jnp.stack([att_src1, att_dst1], axis=-1)
    T2 = jnp.stack([att_src2[0], att_dst2[0]], axis=1)

    W1r = W1.reshape(DF, H1, HC).transpose(1, 0, 2)
    hp_flat, at1 = _tc1(x, W1r, B)
    acc1, den1 = _sc1(hp_flat, at1.reshape(H1, N * 2), src, dst)
    h2pre, at2 = _tc2(acc1, den1, b1.reshape(H1, HC), W2.reshape(H1, HC, HC), T2)
    acc2, den2 = _sc2(h2pre, at2.reshape(N * 2), src, dst)
    out = _tc3(acc2, den2, b2.reshape(1, HC), Wc, bc.reshape(1, 1))
    return jnp.squeeze(out)


# speedup vs baseline: 20.5031x; 1.0585x over previous
import functools

import jax
import jax.numpy as jnp
from jax import lax
from jax.experimental import pallas as pl
from jax.experimental.pallas import tpu as pltpu
from jax.experimental.pallas import tpu_sc as plsc

N = 10000
E = 320000
DF = 128
H1 = 8
HC = 64

NC = 2
NS = 16
L = 16

F32 = jnp.float32
I32 = jnp.int32
_PREC = None


_RB = 1000


def _tc1_body(x_ref, w_ref, b_ref, hp_ref, at_ref):
    h = jnp.dot(x_ref[...], w_ref[0], preferred_element_type=F32, precision=_PREC)
    hp_ref[...] = h
    a0 = jnp.sum(h * b_ref[0, :, 0][None, :], axis=1, keepdims=True)
    a1 = jnp.sum(h * b_ref[0, :, 1][None, :], axis=1, keepdims=True)
    at_ref[0] = jnp.concatenate([a0, a1], axis=1)


def _tc1(x, W1, B):
    nb = N // _RB
    return pl.pallas_call(
        _tc1_body,
        grid=(nb, H1),
        in_specs=[
            pl.BlockSpec((_RB, DF), lambda i, h: (i, 0)),
            pl.BlockSpec((1, DF, HC), lambda i, h: (h, 0, 0)),
            pl.BlockSpec((1, HC, 2), lambda i, h: (h, 0, 0)),
        ],
        out_specs=[
            pl.BlockSpec((_RB, HC), lambda i, h: (h * nb + i, 0)),
            pl.BlockSpec((1, _RB, 2), lambda i, h: (h, i, 0)),
        ],
        out_shape=[
            jax.ShapeDtypeStruct((H1 * N, HC), F32),
            jax.ShapeDtypeStruct((H1, N, 2), F32),
        ],
    )(x, W1, B)


def _tc2_body(acc1_ref, den1_ref, b1r_ref, w2_ref, t2_ref, h2_ref, at2_ref):
    total = None
    for h in range(H1):
        d = den1_ref[h, :, 0:1] + 1e-16
        h1h = jnp.maximum(acc1_ref[h] / d + b1r_ref[h], 0.0)
        t = jnp.dot(h1h, w2_ref[h], preferred_element_type=F32, precision=_PREC)
        total = t if total is None else total + t
    h2_ref[...] = total
    a0 = jnp.sum(total * t2_ref[:, 0][None, :], axis=1, keepdims=True)
    a1 = jnp.sum(total * t2_ref[:, 1][None, :], axis=1, keepdims=True)
    at2_ref[...] = jnp.concatenate([a0, a1], axis=1)


def _tc2(acc1, den1, b1r, W2r, T2):
    nb = N // _RB
    return pl.pallas_call(
        _tc2_body,
        grid=(nb,),
        in_specs=[
            pl.BlockSpec((H1, _RB, HC), lambda i: (0, i, 0)),
            pl.BlockSpec((H1, _RB, 16), lambda i: (0, i, 0)),
            pl.BlockSpec((H1, HC), lambda i: (0, 0)),
            pl.BlockSpec((H1, HC, HC), lambda i: (0, 0, 0)),
            pl.BlockSpec((HC, 2), lambda i: (0, 0)),
        ],
        out_specs=[
            pl.BlockSpec((_RB, HC), lambda i: (i, 0)),
            pl.BlockSpec((_RB, 2), lambda i: (i, 0)),
        ],
        out_shape=[
            jax.ShapeDtypeStruct((N, HC), F32),
            jax.ShapeDtypeStruct((N, 2), F32),
        ],
    )(acc1, den1, b1r, W2r, T2)


def _tc3_body(acc_ref, den_ref, b2_ref, wc_ref, bc_ref, o_ref):
    a = acc_ref[0] + acc_ref[1]
    d = den_ref[0, :, 0:1] + den_ref[1, :, 0:1] + 1e-16
    h2 = jnp.maximum(a / d + b2_ref[...], 0.0)
    o_ref[...] = jnp.dot(h2, wc_ref[...], preferred_element_type=F32, precision=_PREC) + bc_ref[...]


def _tc3(acc2, den2, b2r, Wc, bcr):
    nb = N // _RB
    return pl.pallas_call(
        _tc3_body,
        grid=(nb,),
        in_specs=[
            pl.BlockSpec((2, _RB, HC), lambda i: (0, i, 0)),
            pl.BlockSpec((2, _RB, 16), lambda i: (0, i, 0)),
            pl.BlockSpec((1, HC), lambda i: (0, 0)),
            pl.BlockSpec((HC, 1), lambda i: (0, 0)),
            pl.BlockSpec((1, 1), lambda i: (0, 0)),
        ],
        out_specs=pl.BlockSpec((_RB, 1), lambda i: (i, 0)),
        out_shape=jax.ShapeDtypeStruct((N, 1), F32),
    )(acc2, den2, b2r, Wc, bcr)


_C1 = 160
_EPT1 = E // NS
_NROW = 640


def _sc1_body(hp_ref, at1_ref, sd_ref, acc_o, den_o,
              acc, den, table, idxsd0, idxsd1, idxg0, idxg1,
              rows0, rows1, wbuf0, wbuf1, wflat, sem0, sem1, tsem0, tsem1):
    idxsd = (idxsd0, idxsd1)
    idxg = (idxg0, idxg1)
    rows = (rows0, rows1)
    wbuf2 = (wbuf0, wbuf1)
    wbuf = wbuf0
    sems = (sem0, sem1)
    tsems = (tsem0, tsem1)
    c = lax.axis_index("c")
    s = lax.axis_index("s")
    zf = jnp.zeros((L,), F32)
    io16 = lax.iota(I32, L)

    base_n = s * _NROW
    last = NS - 1

    for r in range(4):
        H = 4 * c + r
        pltpu.sync_copy(at1_ref.at[H], table)

        def zrow(rr, _):
            for q in range(4):
                rows0[rr, pl.ds(16 * q, L)] = zf
            return 0
        lax.fori_loop(0, _C1, zrow, 0)

        def zw(rr, _):
            wbuf[rr, :] = zf
            return 0
        lax.fori_loop(0, _C1, zw, 0)

        def zcp(r0, cnt):
            pltpu.sync_copy(rows0.at[pl.ds(0, cnt)], acc.at[pl.ds(r0, cnt)])
            pltpu.sync_copy(wbuf.at[pl.ds(0, cnt)], den.at[pl.ds(r0, cnt)])

        for k in range(4):
            if k < 2:
                zcp(base_n + 160 * k, 160)
            else:
                pl.when(s < last)(functools.partial(zcp, base_n + 160 * k, 160))
        pl.when(s == last)(functools.partial(zcp, N - 80, 80))
        plsc.subcore_barrier()

        offH = H * N
        nch = _EPT1 // _C1

        def drain_scatter(b):
            pltpu.make_async_copy(hp_ref.at[pl.ds(0, _C1)], rows[b], tsems[b]).wait()
            pltpu.make_async_copy(den_o.at[0].at[pl.ds(0, _C1)], wbuf2[b], tsems[b]).wait()

        def load_chunk(i, o, drain):
            if drain:
                drain_scatter(o)
            k = s * (_EPT1 // _C1) + i
            pltpu.sync_copy(sd_ref.at[k], idxsd[o])

            def f(j, _):
                sv = idxsd[o][0, pl.ds(16 * j, L)]
                idxg[o][pl.ds(16 * j, L)] = sv + offH
                return 0
            lax.fori_loop(0, _C1 // L, f, 0)
            pltpu.async_copy(hp_ref.at[idxg[o]], rows[o], sems[o])

        def process(b):
            def g(j, _):
                sv = idxsd[b][0, pl.ds(16 * j, L)]
                dv = idxsd[b][1, pl.ds(16 * j, L)]
                a_s = plsc.load_gather(table, [sv * 2])
                a_d = plsc.load_gather(table, [dv * 2 + 1])
                e = a_s + a_d
                e = jnp.where(e >= 0.0, e, 0.2 * e)
                wflat[pl.ds(16 * j, L)] = jnp.exp(e)
                return 0
            lax.fori_loop(0, _C1 // L, g, 0)

            pltpu.make_async_copy(hp_ref.at[pl.ds(0, _C1)], rows[b], sems[b]).wait()

            def m(j, _):
                w0v = wflat[pl.ds(16 * j, L)]
                for t in range(L):
                    e = 16 * j + t
                    w0 = w0v[t]
                    for q in range(4):
                        rows[b][e, pl.ds(16 * q, L)] = (
                            rows[b][e, pl.ds(16 * q, L)] * w0)
                    wbuf2[b][e, :] = jnp.where(io16 == 0, w0, 0.0)
                return 0
            lax.fori_loop(0, _C1 // L, m, 0)

            pltpu.async_copy(rows[b], acc.at[idxsd[b].at[1]], tsems[b], add=True)
            pltpu.async_copy(wbuf2[b], den.at[idxsd[b].at[1]], tsems[b], add=True)

        load_chunk(0, 0, False)
        load_chunk(1, 1, False)
        process(0)

        def pipe(gi, _):
            load_chunk(2 * gi + 2, 0, True)
            process(1)
            process(0)
            load_chunk(2 * gi + 3, 1, True)
            return 0
        lax.fori_loop(0, (nch - 3) // 2, pipe, 0)
        load_chunk(nch - 1, 0, True)
        process(1)
        process(0)
        drain_scatter(0)
        drain_scatter(1)
        plsc.subcore_barrier()

        def wslice(r0, cnt):
            pltpu.sync_copy(acc.at[pl.ds(r0, cnt)], acc_o.at[H].at[pl.ds(r0, cnt)])
            pltpu.sync_copy(den.at[pl.ds(r0, cnt)], den_o.at[H].at[pl.ds(r0, cnt)])

        for k in range(4):
            if k < 2:
                wslice(base_n + 160 * k, 160)
            else:
                pl.when(s < last)(functools.partial(wslice, base_n + 160 * k, 160))
        pl.when(s == last)(functools.partial(wslice, N - 80, 80))


def _sc1(hp_flat, at1, sd1):
    mesh = plsc.VectorSubcoreMesh(core_axis_name="c", subcore_axis_name="s",
                                  num_cores=NC, num_subcores=NS)
    f = pl.kernel(
        _sc1_body,
        out_type=(
            jax.ShapeDtypeStruct((H1, N, HC), F32),
            jax.ShapeDtypeStruct((H1, N, 16), F32),
        ),
        mesh=mesh,
        scratch_types=[
            pltpu.VMEM_SHARED((N, HC), F32),
            pltpu.VMEM_SHARED((N, 16), F32),
            pltpu.VMEM((N * 2,), F32),
            pltpu.VMEM((2, _C1), I32),
            pltpu.VMEM((2, _C1), I32),
            pltpu.VMEM((_C1,), I32),
            pltpu.VMEM((_C1,), I32),
            pltpu.VMEM((_C1, HC), F32),
            pltpu.VMEM((_C1, HC), F32),
            pltpu.VMEM((_C1, 16), F32),
            pltpu.VMEM((_C1, 16), F32),
            pltpu.VMEM((_C1,), F32),
            pltpu.SemaphoreType.DMA,
            pltpu.SemaphoreType.DMA,
            pltpu.SemaphoreType.DMA,
            pltpu.SemaphoreType.DMA,
        ],
        compiler_params=pltpu.CompilerParams(needs_layout_passes=False, use_tc_tiling_on_sc=False),
    )
    return f(hp_flat, at1, sd1)


_C2 = 80
_EPT2 = E // (NC * NS)


def _sc2_body(h2_ref, at2_ref, sd_ref, acc_o, den_o,
              acc, den, table, idxsd0, idxsd1,
              rows0, rows1, wbuf0, wbuf1, wflat, sem0, sem1, tsem0, tsem1):
    idxsd = (idxsd0, idxsd1)
    rows = (rows0, rows1)
    wbuf2 = (wbuf0, wbuf1)
    wbuf = wbuf0
    sems = (sem0, sem1)
    tsems = (tsem0, tsem1)
    c = lax.axis_index("c")
    s = lax.axis_index("s")
    zf = jnp.zeros((L,), F32)
    io16 = lax.iota(I32, L)

    pltpu.sync_copy(at2_ref, table)

    def zrow(r, _):
        for q in range(4):
            rows0[r, pl.ds(16 * q, L)] = zf
        return 0
    lax.fori_loop(0, _C2, zrow, 0)

    def zw(r, _):
        wbuf[r, :] = zf
        return 0
    lax.fori_loop(0, _C2, zw, 0)

    base_n = s * _NROW
    last = NS - 1

    def zslice(r0, cnt):
        for k in range(cnt // _C2):
            pltpu.sync_copy(rows0, acc.at[pl.ds(r0 + k * _C2, _C2)])
            pltpu.sync_copy(wbuf, den.at[pl.ds(r0 + k * _C2, _C2)])

    zslice(base_n, 320)
    pl.when(s < last)(functools.partial(zslice, base_n + 320, 320))
    pl.when(s == last)(functools.partial(zslice, N - 80, 80))
    plsc.subcore_barrier()

    wid = c * NS + s
    nch = _EPT2 // _C2

    def drain_scatter(b):
        pltpu.make_async_copy(h2_ref.at[pl.ds(0, _C2)], rows[b], tsems[b]).wait()
        pltpu.make_async_copy(den_o.at[0].at[pl.ds(0, _C2)], wbuf2[b], tsems[b]).wait()

    def load_chunk(i, o, drain):
        if drain:
            drain_scatter(o)
        k = wid * (_EPT2 // _C2) + i
        pltpu.sync_copy(sd_ref.at[k], idxsd[o])
        pltpu.async_copy(h2_ref.at[idxsd[o].at[0]], rows[o], sems[o])

    def process(b):
        def g(j, _):
            sv = idxsd[b][0, pl.ds(16 * j, L)]
            dv = idxsd[b][1, pl.ds(16 * j, L)]
            a_s = plsc.load_gather(table, [sv * 2])
            a_d = plsc.load_gather(table, [dv * 2 + 1])
            e = a_s + a_d
            e = jnp.where(e >= 0.0, e, 0.2 * e)
            wflat[pl.ds(16 * j, L)] = jnp.exp(e)
            return 0
        lax.fori_loop(0, _C2 // L, g, 0)

        pltpu.make_async_copy(h2_ref.at[pl.ds(0, _C2)], rows[b], sems[b]).wait()

        def m(j, _):
            w0v = wflat[pl.ds(16 * j, L)]
            for t in range(L):
                e = 16 * j + t
                w0 = w0v[t]
                for q in range(4):
                    rows[b][e, pl.ds(16 * q, L)] = rows[b][e, pl.ds(16 * q, L)] * w0
                wbuf2[b][e, :] = jnp.where(io16 == 0, w0, 0.0)
            return 0
        lax.fori_loop(0, _C2 // L, m, 0)

        pltpu.async_copy(rows[b], acc.at[idxsd[b].at[1]], tsems[b], add=True)
        pltpu.async_copy(wbuf2[b], den.at[idxsd[b].at[1]], tsems[b], add=True)

    load_chunk(0, 0, False)
    load_chunk(1, 1, False)
    process(0)

    def pipe(gi, _):
        load_chunk(2 * gi + 2, 0, True)
        process(1)
        process(0)
        load_chunk(2 * gi + 3, 1, True)
        return 0
    lax.fori_loop(0, (nch - 3) // 2, pipe, 0)
    load_chunk(nch - 1, 0, True)
    process(1)
    process(0)
    drain_scatter(0)
    drain_scatter(1)
    plsc.subcore_barrier()

    def wslice(r0, cnt):
        pltpu.sync_copy(acc.at[pl.ds(r0, cnt)], acc_o.at[c].at[pl.ds(r0, cnt)])
        pltpu.sync_copy(den.at[pl.ds(r0, cnt)], den_o.at[c].at[pl.ds(r0, cnt)])

    wslice(base_n, 320)
    pl.when(s < last)(functools.partial(wslice, base_n + 320, 320))
    pl.when(s == last)(functools.partial(wslice, N - 80, 80))


def _sc2(h2pre, at2, sd2):
    mesh = plsc.VectorSubcoreMesh(core_axis_name="c", subcore_axis_name="s",
                                  num_cores=NC, num_subcores=NS)
    f = pl.kernel(
        _sc2_body,
        out_type=(
            jax.ShapeDtypeStruct((2, N, HC), F32),
            jax.ShapeDtypeStruct((2, N, 16), F32),
        ),
        mesh=mesh,
        scratch_types=[
            pltpu.VMEM_SHARED((N, HC), F32),
            pltpu.VMEM_SHARED((N, 16), F32),
            pltpu.VMEM((N * 2,), F32),
            pltpu.VMEM((2, _C2), I32),
            pltpu.VMEM((2, _C2), I32),
            pltpu.VMEM((_C2, HC), F32),
            pltpu.VMEM((_C2, HC), F32),
            pltpu.VMEM((_C2, 16), F32),
            pltpu.VMEM((_C2, 16), F32),
            pltpu.VMEM((_C2,), F32),
            pltpu.SemaphoreType.DMA,
            pltpu.SemaphoreType.DMA,
            pltpu.SemaphoreType.DMA,
            pltpu.SemaphoreType.DMA,
        ],
        compiler_params=pltpu.CompilerParams(needs_layout_passes=False, use_tc_tiling_on_sc=False),
    )
    return f(h2pre, at2, sd2)


def kernel(x, edge_index, W1, att_src1, att_dst1, b1,
           W2, att_src2, att_dst2, b2, Wc, bc):
    src = edge_index[0].astype(I32)
    dst = edge_index[1].astype(I32)

    B = jnp.stack([att_src1, att_dst1], axis=-1)
    T2 = jnp.stack([att_src2[0], att_dst2[0]], axis=1)

    W1r = W1.reshape(DF, H1, HC).transpose(1, 0, 2)
    sd1 = jnp.stack([src.reshape(E // _C1, _C1), dst.reshape(E // _C1, _C1)], axis=1)
    sd2 = jnp.stack([src.reshape(E // _C2, _C2), dst.reshape(E // _C2, _C2)], axis=1)

    hp_flat, at1 = _tc1(x, W1r, B)
    acc1, den1 = _sc1(hp_flat, at1.reshape(H1, N * 2), sd1)
    h2pre, at2 = _tc2(acc1, den1, b1.reshape(H1, HC), W2.reshape(H1, HC, HC), T2)
    acc2, den2 = _sc2(h2pre, at2.reshape(N * 2), sd2)
    out = _tc3(acc2, den2, b2.reshape(1, HC), Wc, bc.reshape(1, 1))
    return jnp.squeeze(out)
